# Initial kernel scaffold; baseline (speedup 1.0000x reference)
import jax
import jax.numpy as jnp
from jax.experimental import pallas as pl


def kernel(x, edge_index, edge_attr, batch, bn_weight, bn_bias, bn_mean, bn_var, Wl, bl, Wr, br, We, att, conv_bias, lin_W, lin_b):
    raise NotImplementedError("write your pallas kernel here")


# speedup vs baseline: 8.9443x; 8.9443x over previous
import functools

import jax
import jax.numpy as jnp
from jax import lax
from jax.experimental import pallas as pl
from jax.experimental.pallas import tpu as pltpu
from jax.experimental.pallas import tpu_sc as plsc

N = 10000
E = 320000
D = 128
H = 10
C = 64
G = 64
HC = H * C

NP = 10240
RPS = NP // 16
NW = 32
EPT = E // NW
B = 16
NBLK = EPT // B

_MESH = plsc.VectorSubcoreMesh(core_axis_name="c", subcore_axis_name="s")

_GDN = lax.GatherDimensionNumbers(
    offset_dims=(), collapsed_slice_dims=(0,), start_index_map=(0,))


def _lperm(v, idx):
    return lax.gather(v, idx[:, None], _GDN, (1,),
                      mode=lax.GatherScatterMode.PROMISE_IN_BOUNDS)


def _allsum16(v, lane):
    for sh in (8, 4, 2, 1):
        v = v + _lperm(v, lane ^ sh)
    return v


def _stage_idx(idx1, idx2):
    idx2[0, pl.ds(0, 16)] = idx1[pl.ds(0, 16)]


@functools.partial(
    pl.kernel,
    out_type=jax.ShapeDtypeStruct((2, NP, D), jnp.float32),
    mesh=_MESH,
    scratch_types=[
        pltpu.VMEM((B,), jnp.int32),
        pltpu.VMEM((1, B), jnp.int32),
        pltpu.VMEM((B, D), jnp.float32),
        pltpu.VMEM_SHARED((NP, D), jnp.float32),
    ],
)
def _sc_ea_scatter(dst_hbm, ea_hbm, z128_hbm,
                   easum_out,
                   dst1, dst2, ea_v, ea_sh):
    cid = lax.axis_index("c")
    sid = lax.axis_index("s")
    wid = sid * 2 + cid
    r0 = sid * RPS
    pltpu.sync_copy(z128_hbm.at[pl.ds(r0, RPS)], ea_sh.at[pl.ds(r0, RPS)])
    plsc.subcore_barrier()

    def blk(j, _):
        base = wid * EPT + j * B
        pltpu.sync_copy(dst_hbm.at[pl.ds(base, B)], dst1)
        _stage_idx(dst1, dst2)
        pltpu.sync_copy(ea_hbm.at[pl.ds(base, B)], ea_v)
        pltpu.sync_copy(ea_v, ea_sh.at[dst2.at[0]], add=True)
        return 0
    lax.fori_loop(0, NBLK, blk, 0)
    plsc.subcore_barrier()

    @pl.when(sid == 0)
    def _():
        pltpu.sync_copy(ea_sh, easum_out.at[cid])


def _a1_body(x_ref, bnw_ref, bnb_ref, bnm_ref, bnv_ref,
             Wl_ref, bl_ref, Wr_ref, br_ref, xl_ref, xr_ref):
    x = x_ref[...]
    h = (x - bnm_ref[...]) / jnp.sqrt(bnv_ref[...] + 1e-5) * bnw_ref[...] + bnb_ref[...]
    xl_ref[...] = jnp.dot(h, Wl_ref[...],
                          preferred_element_type=jnp.float32) + bl_ref[...]
    xr_ref[...] = jnp.dot(h, Wr_ref[...],
                          preferred_element_type=jnp.float32) + br_ref[...]


def _run_a1(x, bnw, bnb, bnm, bnv, Wl, bl2, Wr, br2):
    bn = 1000
    return pl.pallas_call(
        _a1_body,
        grid=(N // bn,),
        in_specs=[
            pl.BlockSpec((bn, D), lambda i: (i, 0)),
            pl.BlockSpec((1, D), lambda i: (0, 0)),
            pl.BlockSpec((1, D), lambda i: (0, 0)),
            pl.BlockSpec((1, D), lambda i: (0, 0)),
            pl.BlockSpec((1, D), lambda i: (0, 0)),
            pl.BlockSpec((D, HC), lambda i: (0, 0)),
            pl.BlockSpec((1, HC), lambda i: (0, 0)),
            pl.BlockSpec((D, HC), lambda i: (0, 0)),
            pl.BlockSpec((1, HC), lambda i: (0, 0)),
        ],
        out_specs=[
            pl.BlockSpec((bn, HC), lambda i: (i, 0)),
            pl.BlockSpec((bn, HC), lambda i: (i, 0)),
        ],
        out_shape=[
            jax.ShapeDtypeStruct((N, HC), jnp.float32),
            jax.ShapeDtypeStruct((N, HC), jnp.float32),
        ],
    )(x, bnw, bnb, bnm, bnv, Wl, bl2, Wr, br2)


def _a2_body(ea_ref, We_ref, ef_ref):
    ef_ref[...] = jnp.dot(ea_ref[...], We_ref[...],
                          preferred_element_type=jnp.float32)


def _run_a2(edge_attr, We):
    be = 2000
    return pl.pallas_call(
        _a2_body,
        grid=(E // be,),
        in_specs=[
            pl.BlockSpec((be, D), lambda i: (i, 0)),
            pl.BlockSpec((D, HC), lambda i: (0, 0)),
        ],
        out_specs=pl.BlockSpec((be, HC), lambda i: (i, 0)),
        out_shape=jax.ShapeDtypeStruct((E, HC), jnp.float32),
    )(edge_attr, We)


@functools.partial(
    pl.kernel,
    out_type=(jax.ShapeDtypeStruct((E, 16), jnp.float32),
              jax.ShapeDtypeStruct((2, NP, 128), jnp.float32)),
    mesh=_MESH,
    scratch_types=[
        pltpu.VMEM((B,), jnp.int32),
        pltpu.VMEM((B,), jnp.int32),
        pltpu.VMEM((1, B), jnp.int32),
        pltpu.VMEM((B, HC), jnp.float32),
        pltpu.VMEM((B, HC), jnp.float32),
        pltpu.VMEM((B, HC), jnp.float32),
        pltpu.VMEM((HC,), jnp.float32),
        pltpu.VMEM((B, 16), jnp.float32),
        pltpu.VMEM((B, 128), jnp.float32),
        pltpu.VMEM_SHARED((NP, 128), jnp.float32),
        pltpu.SemaphoreType.DMA,
        pltpu.SemaphoreType.DMA,
        pltpu.SemaphoreType.DMA,
    ],
)
def _sc_attn(src_hbm, dst_hbm, xl_hbm, xr_hbm, ef_hbm, att_hbm, z128_hbm,
             ex_out, den_out,
             src1, dst1, dst2, xlr, xrr, efr, attv, exb16, exb128, den_sh,
             s1, s2, s3):
    cid = lax.axis_index("c")
    sid = lax.axis_index("s")
    wid = sid * 2 + cid
    r0 = sid * RPS
    pltpu.sync_copy(z128_hbm.at[pl.ds(r0, RPS)], den_sh.at[pl.ds(r0, RPS)])
    pltpu.sync_copy(att_hbm, attv)
    lane = lax.iota(jnp.int32, 16)
    one0 = jnp.where(lane == 0, 1.0, 0.0)
    zero16 = jnp.zeros((16,), jnp.float32)

    def initrow(e, _):
        exb128[e, pl.ds(16, 16)] = one0
        for q in range(2, 8):
            exb128[e, pl.ds(q * 16, 16)] = zero16
        return 0
    lax.fori_loop(0, B, initrow, 0)
    plsc.subcore_barrier()

    def blk(j, _):
        base = wid * EPT + j * B
        pltpu.sync_copy(src_hbm.at[pl.ds(base, B)], src1)
        pltpu.sync_copy(dst_hbm.at[pl.ds(base, B)], dst1)
        _stage_idx(dst1, dst2)
        c1 = pltpu.make_async_copy(xl_hbm.at[src1], xlr, s1)
        c2 = pltpu.make_async_copy(xr_hbm.at[dst1], xrr, s2)
        c3 = pltpu.make_async_copy(ef_hbm.at[pl.ds(base, B)], efr, s3)
        c1.start()
        c2.start()
        c3.start()
        c1.wait()
        c2.wait()
        c3.wait()

        def edge(e, _):
            alpha_vec = jnp.zeros((16,), jnp.float32)
            for h_ in range(H):
                acc = jnp.zeros((16,), jnp.float32)
                for q in range(4):
                    off = h_ * C + q * 16
                    mm = (xlr[e, pl.ds(off, 16)] + xrr[e, pl.ds(off, 16)]
                          + efr[e, pl.ds(off, 16)])
                    mm = jnp.maximum(mm, 0.2 * mm)
                    acc = acc + mm * attv[pl.ds(off, 16)]
                alpha_vec = jnp.where(lane == h_, _allsum16(acc, lane),
                                      alpha_vec)
            exv = jnp.exp(alpha_vec)
            exb16[e, :] = exv
            exb128[e, pl.ds(0, 16)] = exv
            return 0
        lax.fori_loop(0, B, edge, 0)
        pltpu.sync_copy(exb16, ex_out.at[pl.ds(base, B)])
        pltpu.sync_copy(exb128, den_sh.at[dst2.at[0]], add=True)
        return 0
    lax.fori_loop(0, NBLK, blk, 0)
    plsc.subcore_barrier()

    @pl.when(sid == 0)
    def _():
        pltpu.sync_copy(den_sh, den_out.at[cid])


def _c_body(eap_ref, denp_ref, xl_ref, xr_ref, We_ref, att_ref,
            exs_ref, rd_ref):
    cnt = denp_ref[0, :, 16:17] + denp_ref[1, :, 16:17]
    easum = eap_ref[0] + eap_ref[1]
    la = jnp.where(cnt > 0, easum / jnp.maximum(cnt, 1.0), 0.0)
    ms = xl_ref[...] + xr_ref[...] + jnp.dot(la, We_ref[...],
                                             preferred_element_type=jnp.float32)
    ms = jnp.maximum(ms, 0.2 * ms)
    t = ms * att_ref[...]
    sel = (lax.broadcasted_iota(jnp.int32, (HC, 16), 0) // C
           == lax.broadcasted_iota(jnp.int32, (HC, 16), 1)).astype(jnp.float32)
    alpha = jnp.dot(t, sel, preferred_element_type=jnp.float32)
    exs = jnp.exp(alpha)
    den16 = denp_ref[0, :, :16] + denp_ref[1, :, :16]
    rd16 = 1.0 / (den16 + exs + 1e-16)
    exs_ref[...] = exs
    rd_ref[...] = jnp.concatenate(
        [rd16, jnp.zeros((rd16.shape[0], 128 - 16), jnp.float32)], axis=1)


def _run_c(eap, denp, xl, xr, We, att1):
    bn = 1000
    return pl.pallas_call(
        _c_body,
        grid=(N // bn,),
        in_specs=[
            pl.BlockSpec((2, bn, D), lambda i: (0, i, 0)),
            pl.BlockSpec((2, bn, 128), lambda i: (0, i, 0)),
            pl.BlockSpec((bn, HC), lambda i: (i, 0)),
            pl.BlockSpec((bn, HC), lambda i: (i, 0)),
            pl.BlockSpec((D, HC), lambda i: (0, 0)),
            pl.BlockSpec((1, HC), lambda i: (0, 0)),
        ],
        out_specs=[
            pl.BlockSpec((bn, 16), lambda i: (i, 0)),
            pl.BlockSpec((bn, 128), lambda i: (i, 0)),
        ],
        out_shape=[
            jax.ShapeDtypeStruct((N, 16), jnp.float32),
            jax.ShapeDtypeStruct((NP, 128), jnp.float32),
        ],
    )(eap, denp, xl, xr, We, att1)


@functools.partial(
    pl.kernel,
    out_type=jax.ShapeDtypeStruct((2, NP, 128), jnp.float32),
    mesh=_MESH,
    scratch_types=[
        pltpu.VMEM((B,), jnp.int32),
        pltpu.VMEM((B,), jnp.int32),
        pltpu.VMEM((1, B), jnp.int32),
        pltpu.VMEM((B, HC), jnp.float32),
        pltpu.VMEM((B, 128), jnp.float32),
        pltpu.VMEM((B, 16), jnp.float32),
        pltpu.VMEM((B, 128), jnp.float32),
        pltpu.VMEM_SHARED((NP, 128), jnp.float32),
        pltpu.SemaphoreType.DMA,
        pltpu.SemaphoreType.DMA,
    ],
)
def _sc_msg(src_hbm, dst_hbm, xl_hbm, rd_hbm, ex_hbm, z128_hbm,
            acc_out,
            src1, dst1, dst2, xlr, rdr, exbv, sbl, acc_sh, s1, s2):
    cid = lax.axis_index("c")
    sid = lax.axis_index("s")
    wid = sid * 2 + cid
    r0 = sid * RPS
    pltpu.sync_copy(z128_hbm.at[pl.ds(r0, RPS)], acc_sh.at[pl.ds(r0, RPS)])
    zero16 = jnp.zeros((16,), jnp.float32)

    def initrow(e, _):
        for q in range(4, 8):
            sbl[e, pl.ds(q * 16, 16)] = zero16
        return 0
    lax.fori_loop(0, B, initrow, 0)
    plsc.subcore_barrier()

    def blk(j, _):
        base = wid * EPT + j * B
        pltpu.sync_copy(src_hbm.at[pl.ds(base, B)], src1)
        pltpu.sync_copy(dst_hbm.at[pl.ds(base, B)], dst1)
        _stage_idx(dst1, dst2)
        c1 = pltpu.make_async_copy(xl_hbm.at[src1], xlr, s1)
        c2 = pltpu.make_async_copy(rd_hbm.at[dst1], rdr, s2)
        c1.start()
        c2.start()
        pltpu.sync_copy(ex_hbm.at[pl.ds(base, B)], exbv)
        c1.wait()
        c2.wait()

        def edge(e, _):
            av = exbv[e, :] * rdr[e, pl.ds(0, 16)]
            sq = [jnp.zeros((16,), jnp.float32) for _ in range(4)]
            for h_ in range(H):
                a = _lperm(av, jnp.full((16,), h_, jnp.int32))
                for q in range(4):
                    sq[q] = sq[q] + a * xlr[e, pl.ds(h_ * C + q * 16, 16)]
            for q in range(4):
                sbl[e, pl.ds(q * 16, 16)] = sq[q]
            return 0
        lax.fori_loop(0, B, edge, 0)
        pltpu.sync_copy(sbl, acc_sh.at[dst2.at[0]], add=True)
        return 0
    lax.fori_loop(0, NBLK, blk, 0)
    plsc.subcore_barrier()

    @pl.when(sid == 0)
    def _():
        pltpu.sync_copy(acc_sh, acc_out.at[cid])


def _g1_body(xl_ref, exs_ref, rd_ref, accp_ref, batch_ref, cb_ref,
             psum_ref, pcnt_ref):
    i = pl.program_id(0)
    w = exs_ref[...] * rd_ref[:, :16]
    sel_t = (lax.broadcasted_iota(jnp.int32, (16, HC), 0)
             == lax.broadcasted_iota(jnp.int32, (16, HC), 1) // C
             ).astype(jnp.float32)
    wexp = jnp.dot(w, sel_t, preferred_element_type=jnp.float32)
    proj = (lax.broadcasted_iota(jnp.int32, (HC, C), 0) % C
            == lax.broadcasted_iota(jnp.int32, (HC, C), 1)).astype(jnp.float32)
    msg = jnp.dot(xl_ref[...] * wexp, proj, preferred_element_type=jnp.float32)
    node = (accp_ref[0, :, :C] + accp_ref[1, :, :C] + msg) * (1.0 / H) + cb_ref[...]
    node = jnp.maximum(node, 0.0)
    oh = (batch_ref[...] == lax.broadcasted_iota(jnp.int32, (1000, G), 1)
          ).astype(jnp.float32)
    psum_c = lax.dot_general(oh, node, (((0,), (0,)), ((), ())),
                             preferred_element_type=jnp.float32)
    pcnt_c = lax.dot_general(oh, jnp.ones((1000, 8), jnp.float32),
                             (((0,), (0,)), ((), ())),
                             preferred_element_type=jnp.float32)

    @pl.when(i == 0)
    def _():
        psum_ref[...] = jnp.zeros_like(psum_ref)
        pcnt_ref[...] = jnp.zeros_like(pcnt_ref)

    psum_ref[...] += psum_c
    pcnt_ref[...] += pcnt_c


def _run_g1(xl, exs, rd, accp, batch2d, cb1):
    bn = 1000
    return pl.pallas_call(
        _g1_body,
        grid=(N // bn,),
        in_specs=[
            pl.BlockSpec((bn, HC), lambda i: (i, 0)),
            pl.BlockSpec((bn, 16), lambda i: (i, 0)),
            pl.BlockSpec((bn, 128), lambda i: (i, 0)),
            pl.BlockSpec((2, bn, 128), lambda i: (0, i, 0)),
            pl.BlockSpec((bn, 1), lambda i: (i, 0)),
            pl.BlockSpec((1, C), lambda i: (0, 0)),
        ],
        out_specs=[
            pl.BlockSpec((G, C), lambda i: (0, 0)),
            pl.BlockSpec((G, 8), lambda i: (0, 0)),
        ],
        out_shape=[
            jax.ShapeDtypeStruct((G, C), jnp.float32),
            jax.ShapeDtypeStruct((G, 8), jnp.float32),
        ],
    )(xl, exs, rd, accp, batch2d, cb1)


def _g2_body(psum_ref, pcnt_ref, lw_ref, lb_ref, out_ref):
    c = pcnt_ref[:, 0:1]
    pooled = jnp.where(c > 0, psum_ref[...] / jnp.maximum(c, 1.0), 0.0)
    out_ref[...] = jnp.dot(pooled, lw_ref[...],
                           preferred_element_type=jnp.float32) + lb_ref[...]


def _run_g2(psum, pcnt, lwp, lbp):
    return pl.pallas_call(
        _g2_body,
        in_specs=[
            pl.BlockSpec((G, C), lambda: (0, 0)),
            pl.BlockSpec((G, 8), lambda: (0, 0)),
            pl.BlockSpec((C, 128), lambda: (0, 0)),
            pl.BlockSpec((1, 128), lambda: (0, 0)),
        ],
        out_specs=pl.BlockSpec((G, 128), lambda: (0, 0)),
        out_shape=jax.ShapeDtypeStruct((G, 128), jnp.float32),
    )(psum, pcnt, lwp, lbp)


def kernel(x, edge_index, edge_attr, batch, bn_weight, bn_bias, bn_mean,
           bn_var, Wl, bl, Wr, br, We, att, conv_bias, lin_W, lin_b):
    src = edge_index[0]
    dst = edge_index[1]
    batch2d = batch.reshape(N, 1)
    bnw = bn_weight.reshape(1, D)
    bnb = bn_bias.reshape(1, D)
    bnm = bn_mean.reshape(1, D)
    bnv = bn_var.reshape(1, D)
    bl2 = bl.reshape(1, HC)
    br2 = br.reshape(1, HC)
    att1 = att.reshape(1, HC)
    attf = att.reshape(HC)
    cb1 = conv_bias.reshape(1, C)
    lwp = jnp.pad(lin_W, ((0, 0), (0, 128 - 2)))
    lbp = jnp.pad(lin_b, (0, 128 - 2)).reshape(1, 128)
    z128 = jnp.zeros((NP, 128), jnp.float32)

    eap = _sc_ea_scatter(dst, edge_attr, z128)
    xl, xr = _run_a1(x, bnw, bnb, bnm, bnv, Wl, bl2, Wr, br2)
    ef = _run_a2(edge_attr, We)
    ex, denp = _sc_attn(src, dst, xl, xr, ef, attf, z128)
    exs, rd = _run_c(eap, denp, xl, xr, We, att1)
    accp = _sc_msg(src, dst, xl, rd, ex, z128)
    psum, pcnt = _run_g1(xl, exs, rd, accp, batch2d, cb1)
    outp = _run_g2(psum, pcnt, lwp, lbp)
    return outp[:, :2]


# speedup vs baseline: 12.0999x; 1.3528x over previous
import functools

import jax
import jax.numpy as jnp
from jax import lax
from jax.experimental import pallas as pl
from jax.experimental.pallas import tpu as pltpu
from jax.experimental.pallas import tpu_sc as plsc

N = 10000
E = 320000
D = 128
H = 10
C = 64
G = 64
HC = H * C

NP = 10240
RPS = NP // 16
NW = 32
EPT = E // NW
B = 16
NBLK = EPT // B

_MESH = plsc.VectorSubcoreMesh(core_axis_name="c", subcore_axis_name="s")

_GDN = lax.GatherDimensionNumbers(
    offset_dims=(), collapsed_slice_dims=(0,), start_index_map=(0,))


def _lperm(v, idx):
    return lax.gather(v, idx[:, None], _GDN, (1,),
                      mode=lax.GatherScatterMode.PROMISE_IN_BOUNDS)


def _allsum16(v, lane):
    for sh in (8, 4, 2, 1):
        v = v + _lperm(v, lane ^ sh)
    return v


def _stage_idx(idx1, idx2):
    idx2[0, pl.ds(0, 16)] = idx1[pl.ds(0, 16)]


@functools.partial(
    pl.kernel,
    out_type=jax.ShapeDtypeStruct((2, NP, D), jnp.float32),
    mesh=_MESH,
    scratch_types=[
        pltpu.VMEM((B,), jnp.int32),
        pltpu.VMEM((1, B), jnp.int32),
        pltpu.VMEM((B, D), jnp.float32),
        pltpu.VMEM_SHARED((NP, D), jnp.float32),
    ],
)
def _sc_ea_scatter(dst_hbm, ea_hbm, z128_hbm,
                   easum_out,
                   dst1, dst2, ea_v, ea_sh):
    cid = lax.axis_index("c")
    sid = lax.axis_index("s")
    wid = sid * 2 + cid
    r0 = sid * RPS
    pltpu.sync_copy(z128_hbm.at[pl.ds(r0, RPS)], ea_sh.at[pl.ds(r0, RPS)])
    plsc.subcore_barrier()

    def blk(j, _):
        base = wid * EPT + j * B
        pltpu.sync_copy(dst_hbm.at[pl.ds(base, B)], dst1)
        _stage_idx(dst1, dst2)
        pltpu.sync_copy(ea_hbm.at[pl.ds(base, B)], ea_v)
        pltpu.sync_copy(ea_v, ea_sh.at[dst2.at[0]], add=True)
        return 0
    lax.fori_loop(0, NBLK, blk, 0)
    plsc.subcore_barrier()

    @pl.when(sid == 0)
    def _():
        pltpu.sync_copy(ea_sh, easum_out.at[cid])


def _a1_body(x_ref, bnw_ref, bnb_ref, bnm_ref, bnv_ref,
             Wl_ref, bl_ref, Wr_ref, br_ref, xl_ref, xr_ref):
    x = x_ref[...]
    h = (x - bnm_ref[...]) / jnp.sqrt(bnv_ref[...] + 1e-5) * bnw_ref[...] + bnb_ref[...]
    xl_ref[...] = jnp.dot(h, Wl_ref[...],
                          preferred_element_type=jnp.float32) + bl_ref[...]
    xr_ref[...] = jnp.dot(h, Wr_ref[...],
                          preferred_element_type=jnp.float32) + br_ref[...]


def _run_a1(x, bnw, bnb, bnm, bnv, Wl, bl2, Wr, br2):
    bn = 1000
    return pl.pallas_call(
        _a1_body,
        grid=(N // bn,),
        in_specs=[
            pl.BlockSpec((bn, D), lambda i: (i, 0)),
            pl.BlockSpec((1, D), lambda i: (0, 0)),
            pl.BlockSpec((1, D), lambda i: (0, 0)),
            pl.BlockSpec((1, D), lambda i: (0, 0)),
            pl.BlockSpec((1, D), lambda i: (0, 0)),
            pl.BlockSpec((D, HC), lambda i: (0, 0)),
            pl.BlockSpec((1, HC), lambda i: (0, 0)),
            pl.BlockSpec((D, HC), lambda i: (0, 0)),
            pl.BlockSpec((1, HC), lambda i: (0, 0)),
        ],
        out_specs=[
            pl.BlockSpec((bn, HC), lambda i: (i, 0)),
            pl.BlockSpec((bn, HC), lambda i: (i, 0)),
        ],
        out_shape=[
            jax.ShapeDtypeStruct((N, HC), jnp.float32),
            jax.ShapeDtypeStruct((N, HC), jnp.float32),
        ],
    )(x, bnw, bnb, bnm, bnv, Wl, bl2, Wr, br2)


def _a2_body(ea_ref, We_ref, ef_ref):
    ef_ref[...] = jnp.dot(ea_ref[...], We_ref[...],
                          preferred_element_type=jnp.float32)


def _run_a2(edge_attr, We):
    be = 2000
    return pl.pallas_call(
        _a2_body,
        grid=(E // be,),
        in_specs=[
            pl.BlockSpec((be, D), lambda i: (i, 0)),
            pl.BlockSpec((D, HC), lambda i: (0, 0)),
        ],
        out_specs=pl.BlockSpec((be, HC), lambda i: (i, 0)),
        out_shape=jax.ShapeDtypeStruct((E, HC), jnp.float32),
    )(edge_attr, We)


@functools.partial(
    pl.kernel,
    out_type=jax.ShapeDtypeStruct((E, 16), jnp.float32),
    mesh=_MESH,
    scratch_types=[
        pltpu.VMEM((EPT,), jnp.int32),
        pltpu.VMEM((EPT,), jnp.int32),
        pltpu.VMEM((B, HC), jnp.float32),
        pltpu.VMEM((B, HC), jnp.float32),
        pltpu.VMEM((B, HC), jnp.float32),
        pltpu.VMEM((B, HC), jnp.float32),
        pltpu.VMEM((B, HC), jnp.float32),
        pltpu.VMEM((B, HC), jnp.float32),
        pltpu.VMEM((HC,), jnp.float32),
        pltpu.VMEM((B, 16), jnp.float32),
        pltpu.VMEM((B, 16), jnp.float32),
        [pltpu.SemaphoreType.DMA] * 8,
    ],
)
def _sc_attn(src_hbm, dst_hbm, xl_hbm, xr_hbm, ef_hbm, att_hbm,
             ex_out,
             src_all, dst_all, xlr0, xlr1, xrr0, xrr1, efr0, efr1, attv,
             exb0, exb1, sems):
    cid = lax.axis_index("c")
    sid = lax.axis_index("s")
    wid = sid * 2 + cid
    tb = wid * EPT
    pltpu.sync_copy(src_hbm.at[pl.ds(tb, EPT)], src_all)
    pltpu.sync_copy(dst_hbm.at[pl.ds(tb, EPT)], dst_all)
    pltpu.sync_copy(att_hbm, attv)
    lane = lax.iota(jnp.int32, 16)
    xlrs = (xlr0, xlr1)
    xrrs = (xrr0, xrr1)
    efrs = (efr0, efr1)
    exbs = (exb0, exb1)

    def gathers(j, b):
        sv = src_all[pl.ds(j * B, B)]
        dv = dst_all[pl.ds(j * B, B)]
        return (pltpu.make_async_copy(xl_hbm.at[sv], xlrs[b], sems[b]),
                pltpu.make_async_copy(xr_hbm.at[dv], xrrs[b], sems[2 + b]),
                pltpu.make_async_copy(ef_hbm.at[pl.ds(tb + j * B, B)],
                                      efrs[b], sems[4 + b]))

    for b in (0, 1):
        for cp in gathers(jnp.int32(b), b):
            cp.start()

    def process(j, b):
        for cp in gathers(j, b):
            cp.wait()

        @pl.when(j >= 2)
        def _():
            pltpu.make_async_copy(exbs[b], ex_out.at[pl.ds(0, B)],
                                  sems[6 + b]).wait()
        att_regs = [attv[pl.ds(i * 16, 16)] for i in range(40)]

        def edge(e, _):
            alpha_vec = jnp.zeros((16,), jnp.float32)
            for h_ in range(H):
                acc = jnp.zeros((16,), jnp.float32)
                for q in range(4):
                    off = h_ * C + q * 16
                    mm = (xlrs[b][e, pl.ds(off, 16)]
                          + xrrs[b][e, pl.ds(off, 16)]
                          + efrs[b][e, pl.ds(off, 16)])
                    mm = jnp.maximum(mm, 0.2 * mm)
                    acc = acc + mm * att_regs[h_ * 4 + q]
                alpha_vec = jnp.where(lane == h_, _allsum16(acc, lane),
                                      alpha_vec)
            exbs[b][e, :] = jnp.exp(alpha_vec)
            return 0
        lax.fori_loop(0, B, edge, 0)
        pltpu.make_async_copy(exbs[b], ex_out.at[pl.ds(tb + j * B, B)],
                              sems[6 + b]).start()

        @pl.when(j + 2 < NBLK)
        def _():
            for cp in gathers(j + 2, b):
                cp.start()

    def pair(jj, _):
        process(jj * 2, 0)
        process(jj * 2 + 1, 1)
        return 0
    lax.fori_loop(0, NBLK // 2, pair, 0)
    process(jnp.int32(NBLK - 1), 0)
    pltpu.make_async_copy(exb0, ex_out.at[pl.ds(0, B)], sems[6]).wait()
    pltpu.make_async_copy(exb1, ex_out.at[pl.ds(0, B)], sems[7]).wait()


B2 = 80
NB2 = EPT // B2


@functools.partial(
    pl.kernel,
    out_type=jax.ShapeDtypeStruct((2, NP, 128), jnp.float32),
    mesh=_MESH,
    scratch_types=[
        pltpu.VMEM((EPT,), jnp.int32),
        pltpu.VMEM((1, B2), jnp.int32),
        pltpu.VMEM((B2, 16), jnp.float32),
        pltpu.VMEM((B2, 128), jnp.float32),
        pltpu.VMEM_SHARED((NP, 128), jnp.float32),
    ],
)
def _sc_den(dst_hbm, ex_hbm, z128_hbm,
            den_out,
            dst_all, dst2, exb16, exb128, den_sh):
    cid = lax.axis_index("c")
    sid = lax.axis_index("s")
    wid = sid * 2 + cid
    tb = wid * EPT
    r0 = sid * RPS
    pltpu.sync_copy(z128_hbm.at[pl.ds(r0, RPS)], den_sh.at[pl.ds(r0, RPS)])
    pltpu.sync_copy(dst_hbm.at[pl.ds(tb, EPT)], dst_all)
    lane = lax.iota(jnp.int32, 16)
    one0 = jnp.where(lane == 0, 1.0, 0.0)
    zero16 = jnp.zeros((16,), jnp.float32)

    def initrow(e, _):
        exb128[e, pl.ds(16, 16)] = one0
        for q in range(2, 8):
            exb128[e, pl.ds(q * 16, 16)] = zero16
        return 0
    lax.fori_loop(0, B2, initrow, 0)
    plsc.subcore_barrier()

    def blk(j, _):
        pltpu.sync_copy(ex_hbm.at[pl.ds(tb + j * B2, B2)], exb16)

        def row(e, _):
            exb128[e, pl.ds(0, 16)] = exb16[e, :]
            return 0
        lax.fori_loop(0, B2, row, 0)
        for k in range(B2 // 16):
            dst2[0, pl.ds(k * 16, 16)] = dst_all[pl.ds(j * B2 + k * 16, 16)]
        pltpu.sync_copy(exb128, den_sh.at[dst2.at[0]], add=True)
        return 0
    lax.fori_loop(0, NB2, blk, 0)
    plsc.subcore_barrier()

    @pl.when(sid == 0)
    def _():
        pltpu.sync_copy(den_sh, den_out.at[cid])


def _c_body(eap_ref, denp_ref, xl_ref, xr_ref, We_ref, att_ref,
            exs_ref, rd_ref):
    cnt = denp_ref[0, :, 16:17] + denp_ref[1, :, 16:17]
    easum = eap_ref[0] + eap_ref[1]
    la = jnp.where(cnt > 0, easum / jnp.maximum(cnt, 1.0), 0.0)
    ms = xl_ref[...] + xr_ref[...] + jnp.dot(la, We_ref[...],
                                             preferred_element_type=jnp.float32)
    ms = jnp.maximum(ms, 0.2 * ms)
    t = ms * att_ref[...]
    sel = (lax.broadcasted_iota(jnp.int32, (HC, 16), 0) // C
           == lax.broadcasted_iota(jnp.int32, (HC, 16), 1)).astype(jnp.float32)
    alpha = jnp.dot(t, sel, preferred_element_type=jnp.float32)
    exs = jnp.exp(alpha)
    den16 = denp_ref[0, :, :16] + denp_ref[1, :, :16]
    rd16 = 1.0 / (den16 + exs + 1e-16)
    exs_ref[...] = exs
    rd_ref[...] = jnp.concatenate(
        [rd16, jnp.zeros((rd16.shape[0], 128 - 16), jnp.float32)], axis=1)


def _run_c(eap, denp, xl, xr, We, att1):
    bn = 1000
    return pl.pallas_call(
        _c_body,
        grid=(N // bn,),
        in_specs=[
            pl.BlockSpec((2, bn, D), lambda i: (0, i, 0)),
            pl.BlockSpec((2, bn, 128), lambda i: (0, i, 0)),
            pl.BlockSpec((bn, HC), lambda i: (i, 0)),
            pl.BlockSpec((bn, HC), lambda i: (i, 0)),
            pl.BlockSpec((D, HC), lambda i: (0, 0)),
            pl.BlockSpec((1, HC), lambda i: (0, 0)),
        ],
        out_specs=[
            pl.BlockSpec((bn, 16), lambda i: (i, 0)),
            pl.BlockSpec((bn, 128), lambda i: (i, 0)),
        ],
        out_shape=[
            jax.ShapeDtypeStruct((N, 16), jnp.float32),
            jax.ShapeDtypeStruct((NP, 128), jnp.float32),
        ],
    )(eap, denp, xl, xr, We, att1)


@functools.partial(
    pl.kernel,
    out_type=jax.ShapeDtypeStruct((2, NP, 128), jnp.float32),
    mesh=_MESH,
    scratch_types=[
        pltpu.VMEM((B,), jnp.int32),
        pltpu.VMEM((B,), jnp.int32),
        pltpu.VMEM((1, B), jnp.int32),
        pltpu.VMEM((B, HC), jnp.float32),
        pltpu.VMEM((B, 128), jnp.float32),
        pltpu.VMEM((B, 16), jnp.float32),
        pltpu.VMEM((B, 128), jnp.float32),
        pltpu.VMEM_SHARED((NP, 128), jnp.float32),
        pltpu.SemaphoreType.DMA,
        pltpu.SemaphoreType.DMA,
    ],
)
def _sc_msg(src_hbm, dst_hbm, xl_hbm, rd_hbm, ex_hbm, z128_hbm,
            acc_out,
            src1, dst1, dst2, xlr, rdr, exbv, sbl, acc_sh, s1, s2):
    cid = lax.axis_index("c")
    sid = lax.axis_index("s")
    wid = sid * 2 + cid
    r0 = sid * RPS
    pltpu.sync_copy(z128_hbm.at[pl.ds(r0, RPS)], acc_sh.at[pl.ds(r0, RPS)])
    zero16 = jnp.zeros((16,), jnp.float32)

    def initrow(e, _):
        for q in range(4, 8):
            sbl[e, pl.ds(q * 16, 16)] = zero16
        return 0
    lax.fori_loop(0, B, initrow, 0)
    plsc.subcore_barrier()

    def blk(j, _):
        base = wid * EPT + j * B
        pltpu.sync_copy(src_hbm.at[pl.ds(base, B)], src1)
        pltpu.sync_copy(dst_hbm.at[pl.ds(base, B)], dst1)
        _stage_idx(dst1, dst2)
        c1 = pltpu.make_async_copy(xl_hbm.at[src1], xlr, s1)
        c2 = pltpu.make_async_copy(rd_hbm.at[dst1], rdr, s2)
        c1.start()
        c2.start()
        pltpu.sync_copy(ex_hbm.at[pl.ds(base, B)], exbv)
        c1.wait()
        c2.wait()

        def edge(e, _):
            av = exbv[e, :] * rdr[e, pl.ds(0, 16)]
            sq = [jnp.zeros((16,), jnp.float32) for _ in range(4)]
            for h_ in range(H):
                a = _lperm(av, jnp.full((16,), h_, jnp.int32))
                for q in range(4):
                    sq[q] = sq[q] + a * xlr[e, pl.ds(h_ * C + q * 16, 16)]
            for q in range(4):
                sbl[e, pl.ds(q * 16, 16)] = sq[q]
            return 0
        lax.fori_loop(0, B, edge, 0)
        pltpu.sync_copy(sbl, acc_sh.at[dst2.at[0]], add=True)
        return 0
    lax.fori_loop(0, NBLK, blk, 0)
    plsc.subcore_barrier()

    @pl.when(sid == 0)
    def _():
        pltpu.sync_copy(acc_sh, acc_out.at[cid])


def _g1_body(xl_ref, exs_ref, rd_ref, accp_ref, batch_ref, cb_ref,
             psum_ref, pcnt_ref):
    i = pl.program_id(0)
    w = exs_ref[...] * rd_ref[:, :16]
    sel_t = (lax.broadcasted_iota(jnp.int32, (16, HC), 0)
             == lax.broadcasted_iota(jnp.int32, (16, HC), 1) // C
             ).astype(jnp.float32)
    wexp = jnp.dot(w, sel_t, preferred_element_type=jnp.float32)
    proj = (lax.broadcasted_iota(jnp.int32, (HC, C), 0) % C
            == lax.broadcasted_iota(jnp.int32, (HC, C), 1)).astype(jnp.float32)
    msg = jnp.dot(xl_ref[...] * wexp, proj, preferred_element_type=jnp.float32)
    node = (accp_ref[0, :, :C] + accp_ref[1, :, :C] + msg) * (1.0 / H) + cb_ref[...]
    node = jnp.maximum(node, 0.0)
    oh = (batch_ref[...] == lax.broadcasted_iota(jnp.int32, (1000, G), 1)
          ).astype(jnp.float32)
    psum_c = lax.dot_general(oh, node, (((0,), (0,)), ((), ())),
                             preferred_element_type=jnp.float32)
    pcnt_c = lax.dot_general(oh, jnp.ones((1000, 8), jnp.float32),
                             (((0,), (0,)), ((), ())),
                             preferred_element_type=jnp.float32)

    @pl.when(i == 0)
    def _():
        psum_ref[...] = jnp.zeros_like(psum_ref)
        pcnt_ref[...] = jnp.zeros_like(pcnt_ref)

    psum_ref[...] += psum_c
    pcnt_ref[...] += pcnt_c


def _run_g1(xl, exs, rd, accp, batch2d, cb1):
    bn = 1000
    return pl.pallas_call(
        _g1_body,
        grid=(N // bn,),
        in_specs=[
            pl.BlockSpec((bn, HC), lambda i: (i, 0)),
            pl.BlockSpec((bn, 16), lambda i: (i, 0)),
            pl.BlockSpec((bn, 128), lambda i: (i, 0)),
            pl.BlockSpec((2, bn, 128), lambda i: (0, i, 0)),
            pl.BlockSpec((bn, 1), lambda i: (i, 0)),
            pl.BlockSpec((1, C), lambda i: (0, 0)),
        ],
        out_specs=[
            pl.BlockSpec((G, C), lambda i: (0, 0)),
            pl.BlockSpec((G, 8), lambda i: (0, 0)),
        ],
        out_shape=[
            jax.ShapeDtypeStruct((G, C), jnp.float32),
            jax.ShapeDtypeStruct((G, 8), jnp.float32),
        ],
    )(xl, exs, rd, accp, batch2d, cb1)


def _g2_body(psum_ref, pcnt_ref, lw_ref, lb_ref, out_ref):
    c = pcnt_ref[:, 0:1]
    pooled = jnp.where(c > 0, psum_ref[...] / jnp.maximum(c, 1.0), 0.0)
    out_ref[...] = jnp.dot(pooled, lw_ref[...],
                           preferred_element_type=jnp.float32) + lb_ref[...]


def _run_g2(psum, pcnt, lwp, lbp):
    return pl.pallas_call(
        _g2_body,
        in_specs=[
            pl.BlockSpec((G, C), lambda: (0, 0)),
            pl.BlockSpec((G, 8), lambda: (0, 0)),
            pl.BlockSpec((C, 128), lambda: (0, 0)),
            pl.BlockSpec((1, 128), lambda: (0, 0)),
        ],
        out_specs=pl.BlockSpec((G, 128), lambda: (0, 0)),
        out_shape=jax.ShapeDtypeStruct((G, 128), jnp.float32),
    )(psum, pcnt, lwp, lbp)


def kernel(x, edge_index, edge_attr, batch, bn_weight, bn_bias, bn_mean,
           bn_var, Wl, bl, Wr, br, We, att, conv_bias, lin_W, lin_b):
    src = edge_index[0]
    dst = edge_index[1]
    batch2d = batch.reshape(N, 1)
    bnw = bn_weight.reshape(1, D)
    bnb = bn_bias.reshape(1, D)
    bnm = bn_mean.reshape(1, D)
    bnv = bn_var.reshape(1, D)
    bl2 = bl.reshape(1, HC)
    br2 = br.reshape(1, HC)
    att1 = att.reshape(1, HC)
    attf = att.reshape(HC)
    cb1 = conv_bias.reshape(1, C)
    lwp = jnp.pad(lin_W, ((0, 0), (0, 128 - 2)))
    lbp = jnp.pad(lin_b, (0, 128 - 2)).reshape(1, 128)
    z128 = jnp.zeros((NP, 128), jnp.float32)

    eap = _sc_ea_scatter(dst, edge_attr, z128)
    xl, xr = _run_a1(x, bnw, bnb, bnm, bnv, Wl, bl2, Wr, br2)
    ef = _run_a2(edge_attr, We)
    ex = _sc_attn(src, dst, xl, xr, ef, attf)
    denp = _sc_den(dst, ex, z128)
    exs, rd = _run_c(eap, denp, xl, xr, We, att1)
    accp = _sc_msg(src, dst, xl, rd, ex, z128)
    psum, pcnt = _run_g1(xl, exs, rd, accp, batch2d, cb1)
    outp = _run_g2(psum, pcnt, lwp, lbp)
    return outp[:, :2]


# speedup vs baseline: 15.7065x; 1.2981x over previous
import functools

import jax
import jax.numpy as jnp
from jax import lax
from jax.experimental import pallas as pl
from jax.experimental.pallas import tpu as pltpu
from jax.experimental.pallas import tpu_sc as plsc

N = 10000
E = 320000
D = 128
H = 10
C = 64
G = 64
HC = H * C

NP = 10240
RPS = NP // 16
NW = 32
EPT = E // NW
B = 16
NBLK = EPT // B

_MESH = plsc.VectorSubcoreMesh(core_axis_name="c", subcore_axis_name="s")

_GDN = lax.GatherDimensionNumbers(
    offset_dims=(), collapsed_slice_dims=(0,), start_index_map=(0,))


def _lperm(v, idx):
    return lax.gather(v, idx[:, None], _GDN, (1,),
                      mode=lax.GatherScatterMode.PROMISE_IN_BOUNDS)


def _allsum16(v, lane):
    for sh in (8, 4, 2, 1):
        v = v + _lperm(v, lane ^ sh)
    return v


def _stage_idx(idx1, idx2):
    idx2[0, pl.ds(0, 16)] = idx1[pl.ds(0, 16)]


@functools.partial(
    pl.kernel,
    out_type=jax.ShapeDtypeStruct((2, NP, D), jnp.float32),
    mesh=_MESH,
    scratch_types=[
        pltpu.VMEM((B,), jnp.int32),
        pltpu.VMEM((1, B), jnp.int32),
        pltpu.VMEM((B, D), jnp.float32),
        pltpu.VMEM_SHARED((NP, D), jnp.float32),
    ],
)
def _sc_ea_scatter(dst_hbm, ea_hbm, z128_hbm,
                   easum_out,
                   dst1, dst2, ea_v, ea_sh):
    cid = lax.axis_index("c")
    sid = lax.axis_index("s")
    wid = sid * 2 + cid
    r0 = sid * RPS
    pltpu.sync_copy(z128_hbm.at[pl.ds(r0, RPS)], ea_sh.at[pl.ds(r0, RPS)])
    plsc.subcore_barrier()

    def blk(j, _):
        base = wid * EPT + j * B
        pltpu.sync_copy(dst_hbm.at[pl.ds(base, B)], dst1)
        _stage_idx(dst1, dst2)
        pltpu.sync_copy(ea_hbm.at[pl.ds(base, B)], ea_v)
        pltpu.sync_copy(ea_v, ea_sh.at[dst2.at[0]], add=True)
        return 0
    lax.fori_loop(0, NBLK, blk, 0)
    plsc.subcore_barrier()

    @pl.when(sid == 0)
    def _():
        pltpu.sync_copy(ea_sh, easum_out.at[cid])


def _a1_body(x_ref, bnw_ref, bnb_ref, bnm_ref, bnv_ref,
             Wl_ref, bl_ref, Wr_ref, br_ref, xl_ref, xr_ref):
    x = x_ref[...]
    h = (x - bnm_ref[...]) / jnp.sqrt(bnv_ref[...] + 1e-5) * bnw_ref[...] + bnb_ref[...]
    xl_ref[...] = jnp.dot(h, Wl_ref[...],
                          preferred_element_type=jnp.float32) + bl_ref[...]
    xr_ref[...] = jnp.dot(h, Wr_ref[...],
                          preferred_element_type=jnp.float32) + br_ref[...]


def _run_a1(x, bnw, bnb, bnm, bnv, Wl, bl2, Wr, br2):
    bn = 1000
    return pl.pallas_call(
        _a1_body,
        grid=(N // bn,),
        in_specs=[
            pl.BlockSpec((bn, D), lambda i: (i, 0)),
            pl.BlockSpec((1, D), lambda i: (0, 0)),
            pl.BlockSpec((1, D), lambda i: (0, 0)),
            pl.BlockSpec((1, D), lambda i: (0, 0)),
            pl.BlockSpec((1, D), lambda i: (0, 0)),
            pl.BlockSpec((D, HC), lambda i: (0, 0)),
            pl.BlockSpec((1, HC), lambda i: (0, 0)),
            pl.BlockSpec((D, HC), lambda i: (0, 0)),
            pl.BlockSpec((1, HC), lambda i: (0, 0)),
        ],
        out_specs=[
            pl.BlockSpec((bn, HC), lambda i: (i, 0)),
            pl.BlockSpec((bn, HC), lambda i: (i, 0)),
        ],
        out_shape=[
            jax.ShapeDtypeStruct((N, HC), jnp.float32),
            jax.ShapeDtypeStruct((N, HC), jnp.float32),
        ],
    )(x, bnw, bnb, bnm, bnv, Wl, bl2, Wr, br2)


def _a2_body(ea_ref, We_ref, ef_ref):
    ef_ref[...] = jnp.dot(ea_ref[...], We_ref[...],
                          preferred_element_type=jnp.float32)


def _run_a2(edge_attr, We):
    be = 2000
    return pl.pallas_call(
        _a2_body,
        grid=(E // be,),
        in_specs=[
            pl.BlockSpec((be, D), lambda i: (i, 0)),
            pl.BlockSpec((D, HC), lambda i: (0, 0)),
        ],
        out_specs=pl.BlockSpec((be, HC), lambda i: (i, 0)),
        out_shape=jax.ShapeDtypeStruct((E, HC), jnp.float32),
    )(edge_attr, We)


@functools.partial(
    pl.kernel,
    out_type=jax.ShapeDtypeStruct((E, 16), jnp.float32),
    mesh=_MESH,
    scratch_types=[
        pltpu.VMEM((EPT,), jnp.int32),
        pltpu.VMEM((EPT,), jnp.int32),
        pltpu.VMEM((B, HC), jnp.float32),
        pltpu.VMEM((B, HC), jnp.float32),
        pltpu.VMEM((B, HC), jnp.float32),
        pltpu.VMEM((B, HC), jnp.float32),
        pltpu.VMEM((B, HC), jnp.float32),
        pltpu.VMEM((B, HC), jnp.float32),
        pltpu.VMEM((HC,), jnp.float32),
        pltpu.VMEM((B, 16), jnp.float32),
        pltpu.VMEM((B, 16), jnp.float32),
        [pltpu.SemaphoreType.DMA] * 8,
    ],
)
def _sc_attn(src_hbm, dst_hbm, xl_hbm, xr_hbm, ef_hbm, att_hbm,
             ex_out,
             src_all, dst_all, xlr0, xlr1, xrr0, xrr1, efr0, efr1, attv,
             exb0, exb1, sems):
    cid = lax.axis_index("c")
    sid = lax.axis_index("s")
    wid = sid * 2 + cid
    tb = wid * EPT
    pltpu.sync_copy(src_hbm.at[pl.ds(tb, EPT)], src_all)
    pltpu.sync_copy(dst_hbm.at[pl.ds(tb, EPT)], dst_all)
    pltpu.sync_copy(att_hbm, attv)
    lane = lax.iota(jnp.int32, 16)
    xlrs = (xlr0, xlr1)
    xrrs = (xrr0, xrr1)
    efrs = (efr0, efr1)
    exbs = (exb0, exb1)

    def gathers(j, b):
        sv = src_all[pl.ds(j * B, B)]
        dv = dst_all[pl.ds(j * B, B)]
        return (pltpu.make_async_copy(xl_hbm.at[sv], xlrs[b], sems[b]),
                pltpu.make_async_copy(xr_hbm.at[dv], xrrs[b], sems[2 + b]),
                pltpu.make_async_copy(ef_hbm.at[pl.ds(tb + j * B, B)],
                                      efrs[b], sems[4 + b]))

    for b in (0, 1):
        for cp in gathers(jnp.int32(b), b):
            cp.start()

    def process(j, b):
        for cp in gathers(j, b):
            cp.wait()

        @pl.when(j >= 2)
        def _():
            pltpu.make_async_copy(exbs[b], ex_out.at[pl.ds(0, B)],
                                  sems[6 + b]).wait()
        att_regs = [attv[pl.ds(i * 16, 16)] for i in range(40)]

        def edge(e, _):
            alpha_vec = jnp.zeros((16,), jnp.float32)
            for h_ in range(H):
                acc = jnp.zeros((16,), jnp.float32)
                for q in range(4):
                    off = h_ * C + q * 16
                    mm = (xlrs[b][e, pl.ds(off, 16)]
                          + xrrs[b][e, pl.ds(off, 16)]
                          + efrs[b][e, pl.ds(off, 16)])
                    mm = jnp.maximum(mm, 0.2 * mm)
                    acc = acc + mm * att_regs[h_ * 4 + q]
                alpha_vec = jnp.where(lane == h_, _allsum16(acc, lane),
                                      alpha_vec)
            exbs[b][e, :] = jnp.exp(alpha_vec)
            return 0
        lax.fori_loop(0, B, edge, 0)
        pltpu.make_async_copy(exbs[b], ex_out.at[pl.ds(tb + j * B, B)],
                              sems[6 + b]).start()

        @pl.when(j + 2 < NBLK)
        def _():
            for cp in gathers(j + 2, b):
                cp.start()

    def pair(jj, _):
        process(jj * 2, 0)
        process(jj * 2 + 1, 1)
        return 0
    lax.fori_loop(0, NBLK // 2, pair, 0)
    process(jnp.int32(NBLK - 1), 0)
    pltpu.make_async_copy(exb0, ex_out.at[pl.ds(0, B)], sems[6]).wait()
    pltpu.make_async_copy(exb1, ex_out.at[pl.ds(0, B)], sems[7]).wait()


B2 = 80
NB2 = EPT // B2


@functools.partial(
    pl.kernel,
    out_type=jax.ShapeDtypeStruct((2, NP, 128), jnp.float32),
    mesh=_MESH,
    scratch_types=[
        pltpu.VMEM((EPT,), jnp.int32),
        pltpu.VMEM((1, B2), jnp.int32),
        pltpu.VMEM((B2, 16), jnp.float32),
        pltpu.VMEM((B2, 128), jnp.float32),
        pltpu.VMEM_SHARED((NP, 128), jnp.float32),
    ],
)
def _sc_den(dst_hbm, ex_hbm, z128_hbm,
            den_out,
            dst_all, dst2, exb16, exb128, den_sh):
    cid = lax.axis_index("c")
    sid = lax.axis_index("s")
    wid = sid * 2 + cid
    tb = wid * EPT
    r0 = sid * RPS
    pltpu.sync_copy(z128_hbm.at[pl.ds(r0, RPS)], den_sh.at[pl.ds(r0, RPS)])
    pltpu.sync_copy(dst_hbm.at[pl.ds(tb, EPT)], dst_all)
    lane = lax.iota(jnp.int32, 16)
    one0 = jnp.where(lane == 0, 1.0, 0.0)
    zero16 = jnp.zeros((16,), jnp.float32)

    def initrow(e, _):
        exb128[e, pl.ds(16, 16)] = one0
        for q in range(2, 8):
            exb128[e, pl.ds(q * 16, 16)] = zero16
        return 0
    lax.fori_loop(0, B2, initrow, 0)
    plsc.subcore_barrier()

    def blk(j, _):
        pltpu.sync_copy(ex_hbm.at[pl.ds(tb + j * B2, B2)], exb16)

        def row(e, _):
            exb128[e, pl.ds(0, 16)] = exb16[e, :]
            return 0
        lax.fori_loop(0, B2, row, 0)
        for k in range(B2 // 16):
            dst2[0, pl.ds(k * 16, 16)] = dst_all[pl.ds(j * B2 + k * 16, 16)]
        pltpu.sync_copy(exb128, den_sh.at[dst2.at[0]], add=True)
        return 0
    lax.fori_loop(0, NB2, blk, 0)
    plsc.subcore_barrier()

    @pl.when(sid == 0)
    def _():
        pltpu.sync_copy(den_sh, den_out.at[cid])


def _c_body(eap_ref, denp_ref, xl_ref, xr_ref, We_ref, att_ref,
            exs_ref, rd_ref):
    cnt = denp_ref[0, :, 16:17] + denp_ref[1, :, 16:17]
    easum = eap_ref[0] + eap_ref[1]
    la = jnp.where(cnt > 0, easum / jnp.maximum(cnt, 1.0), 0.0)
    ms = xl_ref[...] + xr_ref[...] + jnp.dot(la, We_ref[...],
                                             preferred_element_type=jnp.float32)
    ms = jnp.maximum(ms, 0.2 * ms)
    t = ms * att_ref[...]
    sel = (lax.broadcasted_iota(jnp.int32, (HC, 16), 0) // C
           == lax.broadcasted_iota(jnp.int32, (HC, 16), 1)).astype(jnp.float32)
    alpha = jnp.dot(t, sel, preferred_element_type=jnp.float32)
    exs = jnp.exp(alpha)
    den16 = denp_ref[0, :, :16] + denp_ref[1, :, :16]
    rd16 = 1.0 / (den16 + exs + 1e-16)
    exs_ref[...] = exs
    rd_ref[...] = jnp.concatenate(
        [rd16, jnp.zeros((rd16.shape[0], 128 - 16), jnp.float32)], axis=1)


def _run_c(eap, denp, xl, xr, We, att1):
    bn = 1000
    return pl.pallas_call(
        _c_body,
        grid=(N // bn,),
        in_specs=[
            pl.BlockSpec((2, bn, D), lambda i: (0, i, 0)),
            pl.BlockSpec((2, bn, 128), lambda i: (0, i, 0)),
            pl.BlockSpec((bn, HC), lambda i: (i, 0)),
            pl.BlockSpec((bn, HC), lambda i: (i, 0)),
            pl.BlockSpec((D, HC), lambda i: (0, 0)),
            pl.BlockSpec((1, HC), lambda i: (0, 0)),
        ],
        out_specs=[
            pl.BlockSpec((bn, 16), lambda i: (i, 0)),
            pl.BlockSpec((bn, 128), lambda i: (i, 0)),
        ],
        out_shape=[
            jax.ShapeDtypeStruct((N, 16), jnp.float32),
            jax.ShapeDtypeStruct((NP, 128), jnp.float32),
        ],
    )(eap, denp, xl, xr, We, att1)


@functools.partial(
    pl.kernel,
    out_type=jax.ShapeDtypeStruct((2, NP, 128), jnp.float32),
    mesh=_MESH,
    scratch_types=[
        pltpu.VMEM((EPT,), jnp.int32),
        pltpu.VMEM((1, B), jnp.int32),
        pltpu.VMEM((1, B), jnp.int32),
        pltpu.VMEM((B, HC), jnp.float32),
        pltpu.VMEM((B, HC), jnp.float32),
        pltpu.VMEM((B, 128), jnp.float32),
        pltpu.VMEM((B, 128), jnp.float32),
        pltpu.VMEM((B, 16), jnp.float32),
        pltpu.VMEM((B, 16), jnp.float32),
        pltpu.VMEM((B, 128), jnp.float32),
        pltpu.VMEM((B, 128), jnp.float32),
        pltpu.VMEM_SHARED((NP, 128), jnp.float32),
        [pltpu.SemaphoreType.DMA] * 8,
    ],
)
def _sc_msg(sd_hbm, xl_hbm, rd_hbm, ex_hbm, z128_hbm,
            acc_out,
            sd_all, dst2a, dst2b, xlr0, xlr1, rdr0, rdr1, exb0, exb1,
            sbl0, sbl1, acc_sh, sems):
    cid = lax.axis_index("c")
    sid = lax.axis_index("s")
    wid = sid * 2 + cid
    tb = wid * EPT
    r0 = sid * RPS
    pltpu.sync_copy(z128_hbm.at[pl.ds(r0, RPS)], acc_sh.at[pl.ds(r0, RPS)])
    pltpu.sync_copy(sd_hbm.at[pl.ds(tb, EPT)], sd_all)
    zero16 = jnp.zeros((16,), jnp.float32)
    xlrs = (xlr0, xlr1)
    rdrs = (rdr0, rdr1)
    exbs = (exb0, exb1)
    sbls = (sbl0, sbl1)
    dst2s = (dst2a, dst2b)

    def initrow(e, _):
        for q in range(4, 8):
            sbl0[e, pl.ds(q * 16, 16)] = zero16
            sbl1[e, pl.ds(q * 16, 16)] = zero16
        return 0
    lax.fori_loop(0, B, initrow, 0)
    plsc.subcore_barrier()

    def unpack(j):
        sd = sd_all[pl.ds(j * B, B)]
        return lax.shift_right_logical(sd, 14), lax.bitwise_and(sd, 16383)

    def copies(j, b):
        sv, dv = unpack(j)
        return (pltpu.make_async_copy(xl_hbm.at[sv], xlrs[b], sems[b]),
                pltpu.make_async_copy(rd_hbm.at[dv], rdrs[b], sems[2 + b]),
                pltpu.make_async_copy(ex_hbm.at[pl.ds(tb + j * B, B)],
                                      exbs[b], sems[4 + b]))

    for b in (0, 1):
        for cp in copies(jnp.int32(b), b):
            cp.start()

    def process(j, b):
        for cp in copies(j, b):
            cp.wait()

        def edge(e, _):
            av = exbs[b][e, :] * rdrs[b][e, pl.ds(0, 16)]
            sq = [jnp.zeros((16,), jnp.float32) for _ in range(4)]
            for h_ in range(H):
                a = _lperm(av, jnp.full((16,), h_, jnp.int32))
                for q in range(4):
                    sq[q] = sq[q] + a * xlrs[b][e, pl.ds(h_ * C + q * 16, 16)]
            for q in range(4):
                sbls[b][e, pl.ds(q * 16, 16)] = sq[q]
            return 0
        lax.fori_loop(0, B, edge, 0)
        _, dv = unpack(j)
        dst2s[b][0, pl.ds(0, 16)] = dv
        pltpu.sync_copy(sbls[b], acc_sh.at[dst2s[b].at[0]], add=True)

        @pl.when(j + 2 < NBLK)
        def _():
            for cp in copies(j + 2, b):
                cp.start()

    def pair(jj, _):
        process(jj * 2, 0)
        process(jj * 2 + 1, 1)
        return 0
    lax.fori_loop(0, NBLK // 2, pair, 0)
    process(jnp.int32(NBLK - 1), 0)
    plsc.subcore_barrier()

    @pl.when(sid == 0)
    def _():
        pltpu.sync_copy(acc_sh, acc_out.at[cid])


def _g1_body(xl_ref, exs_ref, rd_ref, accp_ref, batch_ref, cb_ref,
             psum_ref, pcnt_ref):
    i = pl.program_id(0)
    w = exs_ref[...] * rd_ref[:, :16]
    sel_t = (lax.broadcasted_iota(jnp.int32, (16, HC), 0)
             == lax.broadcasted_iota(jnp.int32, (16, HC), 1) // C
             ).astype(jnp.float32)
    wexp = jnp.dot(w, sel_t, preferred_element_type=jnp.float32)
    proj = (lax.broadcasted_iota(jnp.int32, (HC, C), 0) % C
            == lax.broadcasted_iota(jnp.int32, (HC, C), 1)).astype(jnp.float32)
    msg = jnp.dot(xl_ref[...] * wexp, proj, preferred_element_type=jnp.float32)
    node = (accp_ref[0, :, :C] + accp_ref[1, :, :C] + msg) * (1.0 / H) + cb_ref[...]
    node = jnp.maximum(node, 0.0)
    oh = (batch_ref[...] == lax.broadcasted_iota(jnp.int32, (1000, G), 1)
          ).astype(jnp.float32)
    psum_c = lax.dot_general(oh, node, (((0,), (0,)), ((), ())),
                             preferred_element_type=jnp.float32)
    pcnt_c = lax.dot_general(oh, jnp.ones((1000, 8), jnp.float32),
                             (((0,), (0,)), ((), ())),
                             preferred_element_type=jnp.float32)

    @pl.when(i == 0)
    def _():
        psum_ref[...] = jnp.zeros_like(psum_ref)
        pcnt_ref[...] = jnp.zeros_like(pcnt_ref)

    psum_ref[...] += psum_c
    pcnt_ref[...] += pcnt_c


def _run_g1(xl, exs, rd, accp, batch2d, cb1):
    bn = 1000
    return pl.pallas_call(
        _g1_body,
        grid=(N // bn,),
        in_specs=[
            pl.BlockSpec((bn, HC), lambda i: (i, 0)),
            pl.BlockSpec((bn, 16), lambda i: (i, 0)),
            pl.BlockSpec((bn, 128), lambda i: (i, 0)),
            pl.BlockSpec((2, bn, 128), lambda i: (0, i, 0)),
            pl.BlockSpec((bn, 1), lambda i: (i, 0)),
            pl.BlockSpec((1, C), lambda i: (0, 0)),
        ],
        out_specs=[
            pl.BlockSpec((G, C), lambda i: (0, 0)),
            pl.BlockSpec((G, 8), lambda i: (0, 0)),
        ],
        out_shape=[
            jax.ShapeDtypeStruct((G, C), jnp.float32),
            jax.ShapeDtypeStruct((G, 8), jnp.float32),
        ],
    )(xl, exs, rd, accp, batch2d, cb1)


def _g2_body(psum_ref, pcnt_ref, lw_ref, lb_ref, out_ref):
    c = pcnt_ref[:, 0:1]
    pooled = jnp.where(c > 0, psum_ref[...] / jnp.maximum(c, 1.0), 0.0)
    out_ref[...] = jnp.dot(pooled, lw_ref[...],
                           preferred_element_type=jnp.float32) + lb_ref[...]


def _run_g2(psum, pcnt, lwp, lbp):
    return pl.pallas_call(
        _g2_body,
        in_specs=[
            pl.BlockSpec((G, C), lambda: (0, 0)),
            pl.BlockSpec((G, 8), lambda: (0, 0)),
            pl.BlockSpec((C, 128), lambda: (0, 0)),
            pl.BlockSpec((1, 128), lambda: (0, 0)),
        ],
        out_specs=pl.BlockSpec((G, 128), lambda: (0, 0)),
        out_shape=jax.ShapeDtypeStruct((G, 128), jnp.float32),
    )(psum, pcnt, lwp, lbp)


def kernel(x, edge_index, edge_attr, batch, bn_weight, bn_bias, bn_mean,
           bn_var, Wl, bl, Wr, br, We, att, conv_bias, lin_W, lin_b):
    src = edge_index[0]
    dst = edge_index[1]
    batch2d = batch.reshape(N, 1)
    bnw = bn_weight.reshape(1, D)
    bnb = bn_bias.reshape(1, D)
    bnm = bn_mean.reshape(1, D)
    bnv = bn_var.reshape(1, D)
    bl2 = bl.reshape(1, HC)
    br2 = br.reshape(1, HC)
    att1 = att.reshape(1, HC)
    attf = att.reshape(HC)
    cb1 = conv_bias.reshape(1, C)
    lwp = jnp.pad(lin_W, ((0, 0), (0, 128 - 2)))
    lbp = jnp.pad(lin_b, (0, 128 - 2)).reshape(1, 128)
    z128 = jnp.zeros((NP, 128), jnp.float32)

    eap = _sc_ea_scatter(dst, edge_attr, z128)
    xl, xr = _run_a1(x, bnw, bnb, bnm, bnv, Wl, bl2, Wr, br2)
    ef = _run_a2(edge_attr, We)
    ex = _sc_attn(src, dst, xl, xr, ef, attf)
    denp = _sc_den(dst, ex, z128)
    exs, rd = _run_c(eap, denp, xl, xr, We, att1)
    sd = src * jnp.int32(16384) + dst
    accp = _sc_msg(sd, xl, rd, ex, z128)
    psum, pcnt = _run_g1(xl, exs, rd, accp, batch2d, cb1)
    outp = _run_g2(psum, pcnt, lwp, lbp)
    return outp[:, :2]
